# baseline (speedup 1.0000x reference)
import functools

import jax
import jax.numpy as jnp
from jax import lax
from jax.experimental import pallas as pl
from jax.experimental.pallas import tpu as pltpu
from jax.experimental.pallas import tpu_sc as plsc

L = 16
N = 1024


def _body(V, lp_hbm, tgt_hbm, msk_hbm, rew_hbm, out_hbm,
          tgt_v, msk_v, rew_v, idx_v, vals_v, buf_v, out_v, sem):
    c = lax.axis_index("c")
    s = lax.axis_index("s")

    @pl.when(jnp.logical_and(c == 0, s == 0))
    def _all():
        pltpu.sync_copy(tgt_hbm, tgt_v)
        pltpu.sync_copy(msk_hbm, msk_v)
        pltpu.sync_copy(rew_hbm, rew_v)

        for j in range(N // L):
            tgt = tgt_v[pl.ds(j * L, L)]
            n = j * L + lax.iota(jnp.int32, L)
            idx_v[pl.ds(j * L, L)] = n * V + tgt

        pltpu.async_copy(lp_hbm.at[idx_v], vals_v, sem).wait()

        acc = jnp.zeros((L,), jnp.float32)
        cnt = jnp.zeros((L,), jnp.float32)
        for j in range(N // L):
            m = msk_v[pl.ds(j * L, L)]
            acc = acc + vals_v[pl.ds(j * L, L)] * rew_v[pl.ds(j * L, L)] * m
            cnt = cnt + m

        buf_v[pl.ds(L, L)] = jnp.zeros((L,), jnp.float32)

        def lane_reduce(vec):
            cur = vec
            for off in (8, 4, 2, 1):
                buf_v[pl.ds(0, L)] = cur
                cur = cur + buf_v[pl.ds(off, L)]
            return cur

        vsum = lane_reduce(acc)
        csum = lane_reduce(cnt)
        out_v[...] = -(vsum / csum)
        pltpu.sync_copy(out_v, out_hbm)


def kernel(lprobs, target, seq_padding_mask, reward):
    B, S, V = lprobs.shape
    lp_flat = lprobs.reshape(-1)
    tgt = target.reshape(-1).astype(jnp.int32)
    maskf = seq_padding_mask.reshape(-1).astype(jnp.float32)
    rew = reward.reshape(-1)

    mesh = plsc.VectorSubcoreMesh(core_axis_name="c", subcore_axis_name="s")
    run = functools.partial(
        pl.kernel,
        mesh=mesh,
        out_type=jax.ShapeDtypeStruct((L,), jnp.float32),
        scratch_types=[
            pltpu.VMEM((N,), jnp.int32),
            pltpu.VMEM((N,), jnp.float32),
            pltpu.VMEM((N,), jnp.float32),
            pltpu.VMEM((N,), jnp.int32),
            pltpu.VMEM((N,), jnp.float32),
            pltpu.VMEM((2 * L,), jnp.float32),
            pltpu.VMEM((L,), jnp.float32),
            pltpu.SemaphoreType.DMA,
        ],
    )(functools.partial(_body, V))
    out = run(lp_flat, tgt, maskf, rew)
    return out[0]


# speedup vs baseline: 1.0041x; 1.0041x over previous
import functools

import jax
import jax.numpy as jnp
from jax import lax
from jax.experimental import pallas as pl
from jax.experimental.pallas import tpu as pltpu
from jax.experimental.pallas import tpu_sc as plsc

L = 16
NS = 16
TOK = 64
NSTREAM = 8
CH = TOK // NSTREAM


def _body(V, lp_hbm, tgt_hbm, msk_hbm, rew_hbm,
          out_hbm, part_hbm,
          tgt_v, msk_v, rew_v, idx_v, vals_v, pair_v, red_v, buf_v, out_v,
          sem):
    c = lax.axis_index("c")
    s = lax.axis_index("s")

    @pl.when(c == 0)
    def _gather_and_partial():
        base = s * TOK
        pltpu.sync_copy(tgt_hbm.at[pl.ds(base, TOK)], tgt_v)
        pltpu.sync_copy(msk_hbm.at[pl.ds(base, TOK)], msk_v)
        pltpu.sync_copy(rew_hbm.at[pl.ds(base, TOK)], rew_v)

        for j in range(TOK // L):
            tgt = tgt_v[pl.ds(j * L, L)]
            n = base + j * L + lax.iota(jnp.int32, L)
            idx_v[pl.ds(j * L, L)] = n * V + tgt

        handles = [
            pltpu.async_copy(lp_hbm.at[idx_v.at[pl.ds(k * CH, CH)]],
                             vals_v.at[pl.ds(k * CH, CH)], sem)
            for k in range(NSTREAM)
        ]
        for h in handles:
            h.wait()

        acc = jnp.zeros((L,), jnp.float32)
        cnt = jnp.zeros((L,), jnp.float32)
        for j in range(TOK // L):
            m = msk_v[pl.ds(j * L, L)]
            acc = acc + vals_v[pl.ds(j * L, L)] * rew_v[pl.ds(j * L, L)] * m
            cnt = cnt + m
        pair_v[0, :] = acc
        pair_v[1, :] = cnt
        pltpu.sync_copy(pair_v, part_hbm.at[s])

    plsc.subcore_barrier()

    @pl.when(jnp.logical_and(c == 0, s == 0))
    def _finalize():
        pltpu.sync_copy(part_hbm, red_v)
        tot = jnp.zeros((L,), jnp.float32)
        ctot = jnp.zeros((L,), jnp.float32)
        for i in range(NS):
            tot = tot + red_v[i, 0]
            ctot = ctot + red_v[i, 1]

        buf_v[pl.ds(L, L)] = jnp.zeros((L,), jnp.float32)

        def lane_reduce(vec):
            cur = vec
            for off in (8, 4, 2, 1):
                buf_v[pl.ds(0, L)] = cur
                cur = cur + buf_v[pl.ds(off, L)]
            return cur

        vsum = lane_reduce(tot)
        csum = lane_reduce(ctot)
        out_v[...] = -(vsum / csum)
        pltpu.sync_copy(out_v, out_hbm)


def kernel(lprobs, target, seq_padding_mask, reward):
    B, S, V = lprobs.shape
    lp_flat = lprobs.reshape(-1)
    tgt = target.reshape(-1).astype(jnp.int32)
    maskf = seq_padding_mask.reshape(-1).astype(jnp.float32)
    rew = reward.reshape(-1)

    mesh = plsc.VectorSubcoreMesh(core_axis_name="c", subcore_axis_name="s")
    run = functools.partial(
        pl.kernel,
        mesh=mesh,
        out_type=(
            jax.ShapeDtypeStruct((L,), jnp.float32),
            jax.ShapeDtypeStruct((NS, 2, L), jnp.float32),
        ),
        scratch_types=[
            pltpu.VMEM((TOK,), jnp.int32),
            pltpu.VMEM((TOK,), jnp.float32),
            pltpu.VMEM((TOK,), jnp.float32),
            pltpu.VMEM((TOK,), jnp.int32),
            pltpu.VMEM((TOK,), jnp.float32),
            pltpu.VMEM((2, L), jnp.float32),
            pltpu.VMEM((NS, 2, L), jnp.float32),
            pltpu.VMEM((2 * L,), jnp.float32),
            pltpu.VMEM((L,), jnp.float32),
            pltpu.SemaphoreType.DMA,
        ],
    )(functools.partial(_body, V))
    out, _ = run(lp_flat, tgt, maskf, rew)
    return out[0]


# speedup vs baseline: 21.5307x; 21.4422x over previous
import functools

import jax
import jax.numpy as jnp
from jax import lax
from jax.experimental import pallas as pl
from jax.experimental.pallas import tpu as pltpu
from jax.experimental.pallas import tpu_sc as plsc

L = 16
NS = 16
TOK = 64


def _body(V, lp_hbm, tgt_hbm, msk_hbm, rew_hbm, out_hbm, part_hbm,
          tile_v, pair_v, red_v, buf_v, out_v, tgt_v, rew_v, msk_v, sem):
    c = lax.axis_index("c")
    s = lax.axis_index("s")

    @pl.when(c == 0)
    def _gather_and_partial():
        base = s * TOK
        pltpu.sync_copy(tgt_hbm.at[pl.ds(base, TOK)], tgt_v)
        pltpu.sync_copy(rew_hbm.at[pl.ds(base, TOK)], rew_v)
        pltpu.sync_copy(msk_hbm.at[pl.ds(base, TOK)], msk_v)

        handles = []
        for j in range(TOK // L):
            tgtc = tgt_v[pl.ds(j * L, L)]
            for k2 in range(L):
                k = j * L + k2
                n = base + k
                t = tgtc[k2]
                ct = pl.multiple_of((t >> 7) << 7, 128)
                handles.append(
                    pltpu.async_copy(lp_hbm.at[n // 8, :, pl.ds(ct, 128)],
                                     tile_v.at[k], sem))
        for h in handles:
            h.wait()

        lanes = lax.iota(jnp.int32, L)
        acc = jnp.zeros((L,), jnp.float32)
        zero = jnp.zeros((L,), jnp.float32)
        cnt = jnp.zeros((L,), jnp.float32)
        for j in range(TOK // L):
            tgtc = tgt_v[pl.ds(j * L, L)]
            weic = rew_v[pl.ds(j * L, L)] * msk_v[pl.ds(j * L, L)]
            cnt = cnt + msk_v[pl.ds(j * L, L)]
            for k2 in range(L):
                k = j * L + k2
                n = base + k
                t = tgtc[k2]
                c0 = pl.multiple_of(((t & 127) >> 4) << 4, 16)
                off = t & 15
                row = tile_v[k, n % 8, pl.ds(c0, L)]
                sel = jnp.where(lanes == off, row, zero)
                acc = acc + sel * weic[k2]
        pair_v[0, :] = acc
        pair_v[1, :] = cnt
        pltpu.sync_copy(pair_v, part_hbm.at[s])

    plsc.subcore_barrier()

    @pl.when(jnp.logical_and(c == 0, s == 0))
    def _finalize():
        pltpu.sync_copy(part_hbm, red_v)
        tot = jnp.zeros((L,), jnp.float32)
        ctot = jnp.zeros((L,), jnp.float32)
        for i in range(NS):
            tot = tot + red_v[i, 0]
            ctot = ctot + red_v[i, 1]

        buf_v[pl.ds(L, L)] = jnp.zeros((L,), jnp.float32)

        def lane_reduce(vec):
            cur = vec
            for off in (8, 4, 2, 1):
                buf_v[pl.ds(0, L)] = cur
                cur = cur + buf_v[pl.ds(off, L)]
            return cur

        vsum = lane_reduce(tot)
        csum = lane_reduce(ctot)
        out_v[...] = -(vsum / csum)
        pltpu.sync_copy(out_v, out_hbm)


def kernel(lprobs, target, seq_padding_mask, reward):
    B, S, V = lprobs.shape
    lp3 = lprobs.reshape(B * S // 8, 8, V)
    tgt = target.reshape(-1).astype(jnp.int32)
    maskf = seq_padding_mask.reshape(-1).astype(jnp.float32)
    rew = reward.reshape(-1)

    mesh = plsc.VectorSubcoreMesh(core_axis_name="c", subcore_axis_name="s")
    run = functools.partial(
        pl.kernel,
        mesh=mesh,
        out_type=(
            jax.ShapeDtypeStruct((L,), jnp.float32),
            jax.ShapeDtypeStruct((NS, 2, L), jnp.float32),
        ),
        compiler_params=pltpu.CompilerParams(use_tc_tiling_on_sc=True),
        scratch_types=[
            pltpu.VMEM((TOK, 8, 128), jnp.float32),
            pltpu.VMEM((2, L), jnp.float32),
            pltpu.VMEM((NS, 2, L), jnp.float32),
            pltpu.VMEM((2 * L,), jnp.float32),
            pltpu.VMEM((L,), jnp.float32),
            pltpu.VMEM((TOK,), jnp.int32),
            pltpu.VMEM((TOK,), jnp.float32),
            pltpu.VMEM((TOK,), jnp.float32),
            pltpu.SemaphoreType.DMA,
        ],
    )(functools.partial(_body, V))
    out, _ = run(lp3, tgt, maskf, rew)
    return out[0]


# speedup vs baseline: 22.4563x; 1.0430x over previous
import functools

import jax
import jax.numpy as jnp
from jax import lax
from jax.experimental import pallas as pl
from jax.experimental.pallas import tpu as pltpu
from jax.experimental.pallas import tpu_sc as plsc

L = 16
NS = 16
TOK = 64
NSEM = 4


def _body(V, lp_hbm, tgt_hbm, msk_hbm, rew_hbm, out_hbm, part_hbm,
          tile_v, pair_v, red_v, buf_v, out_v,
          tgt_v, rew_v, msk_v, sems):
    s = lax.axis_index("s")
    base = s * TOK

    pltpu.sync_copy(tgt_hbm.at[pl.ds(base, TOK)], tgt_v)
    pltpu.sync_copy(rew_hbm.at[pl.ds(base, TOK)], rew_v)
    pltpu.sync_copy(msk_hbm.at[pl.ds(base, TOK)], msk_v)

    handles = []
    for j in range(TOK // L):
        tgtc = tgt_v[pl.ds(j * L, L)]
        for k2 in range(L):
            k = j * L + k2
            n = base + k
            t = tgtc[k2]
            ct = pl.multiple_of((t >> 7) << 7, 128)
            handles.append(
                pltpu.async_copy(lp_hbm.at[n // 8, :, pl.ds(ct, 128)],
                                 tile_v.at[k], sems[j % NSEM]))

    lanes = lax.iota(jnp.int32, L)
    acc = jnp.zeros((L,), jnp.float32)
    zero = jnp.zeros((L,), jnp.float32)
    cnt = jnp.zeros((L,), jnp.float32)
    for j in range(TOK // L):
        for k2 in range(L):
            handles[j * L + k2].wait()
        tgtc = tgt_v[pl.ds(j * L, L)]
        weic = rew_v[pl.ds(j * L, L)] * msk_v[pl.ds(j * L, L)]
        cnt = cnt + msk_v[pl.ds(j * L, L)]
        for k2 in range(L):
            k = j * L + k2
            n = base + k
            t = tgtc[k2]
            c0 = pl.multiple_of(((t & 127) >> 4) << 4, 16)
            off = t & 15
            row = tile_v[k, n % 8, pl.ds(c0, L)]
            sel = jnp.where(lanes == off, row, zero)
            acc = acc + sel * weic[k2]
    pair_v[0, :] = acc
    pair_v[1, :] = cnt
    pltpu.sync_copy(pair_v, part_hbm.at[s])

    plsc.subcore_barrier()

    @pl.when(s == 0)
    def _finalize():
        pltpu.sync_copy(part_hbm, red_v)
        tot = jnp.zeros((L,), jnp.float32)
        ctot = jnp.zeros((L,), jnp.float32)
        for i in range(NS):
            tot = tot + red_v[i, 0]
            ctot = ctot + red_v[i, 1]

        buf_v[pl.ds(L, L)] = jnp.zeros((L,), jnp.float32)

        def lane_reduce(vec):
            cur = vec
            for off in (8, 4, 2, 1):
                buf_v[pl.ds(0, L)] = cur
                cur = cur + buf_v[pl.ds(off, L)]
            return cur

        vsum = lane_reduce(tot)
        csum = lane_reduce(ctot)
        out_v[...] = -(vsum / csum)
        pltpu.sync_copy(out_v, out_hbm)


def kernel(lprobs, target, seq_padding_mask, reward):
    B, S, V = lprobs.shape
    lp3 = lprobs.reshape(B * S // 8, 8, V)
    tgt = target.reshape(-1).astype(jnp.int32)
    maskf = seq_padding_mask.reshape(-1).astype(jnp.float32)
    rew = reward.reshape(-1)
    mesh = plsc.VectorSubcoreMesh(core_axis_name="c", subcore_axis_name="s",
                                  num_cores=1)
    run = functools.partial(
        pl.kernel,
        mesh=mesh,
        out_type=(
            jax.ShapeDtypeStruct((L,), jnp.float32),
            jax.ShapeDtypeStruct((NS, 2, L), jnp.float32),
        ),
        compiler_params=pltpu.CompilerParams(use_tc_tiling_on_sc=True),
        scratch_types=[
            pltpu.VMEM((TOK, 8, 128), jnp.float32),
            pltpu.VMEM((2, L), jnp.float32),
            pltpu.VMEM((NS, 2, L), jnp.float32),
            pltpu.VMEM((2 * L,), jnp.float32),
            pltpu.VMEM((L,), jnp.float32),
            pltpu.VMEM((TOK,), jnp.int32),
            pltpu.VMEM((TOK,), jnp.float32),
            pltpu.VMEM((TOK,), jnp.float32),
            [pltpu.SemaphoreType.DMA] * NSEM,
        ],
    )(functools.partial(_body, V))
    out, _ = run(lp3, tgt, maskf, rew)
    return out[0]


# speedup vs baseline: 23.0741x; 1.0275x over previous
import functools

import jax
import jax.numpy as jnp
from jax import lax
from jax.experimental import pallas as pl
from jax.experimental.pallas import tpu as pltpu
from jax.experimental.pallas import tpu_sc as plsc

L = 16
NS = 16
TOK = 64
NSEM = 4


def _body(V, lp_hbm, tgt_hbm, msk_hbm, rew_hbm, out_hbm, part_hbm,
          tile_v, pair_v, red_v, buf_v, out_v,
          tgt_v, rew_v, msk_v, sems):
    s = lax.axis_index("s")
    base = s * TOK

    pltpu.sync_copy(tgt_hbm.at[pl.ds(base, TOK)], tgt_v)

    handles = []
    for j in range(TOK // L):
        tgtc = tgt_v[pl.ds(j * L, L)]
        for k2 in range(L):
            k = j * L + k2
            n = base + k
            t = tgtc[k2]
            ct = pl.multiple_of((t >> 7) << 7, 128)
            handles.append(
                pltpu.async_copy(lp_hbm.at[n // 8, :, pl.ds(ct, 128)],
                                 tile_v.at[k], sems[j % NSEM]))

    hr = pltpu.async_copy(rew_hbm.at[pl.ds(base, TOK)], rew_v, sems[NSEM])
    hm = pltpu.async_copy(msk_hbm.at[pl.ds(base, TOK)], msk_v, sems[NSEM])
    hr.wait()
    hm.wait()

    lanes = lax.iota(jnp.int32, L)
    acc = jnp.zeros((L,), jnp.float32)
    zero = jnp.zeros((L,), jnp.float32)
    cnt = jnp.zeros((L,), jnp.float32)
    for j in range(TOK // L):
        for k2 in range(L):
            handles[j * L + k2].wait()
        tgtc = tgt_v[pl.ds(j * L, L)]
        weic = rew_v[pl.ds(j * L, L)] * msk_v[pl.ds(j * L, L)]
        cnt = cnt + msk_v[pl.ds(j * L, L)]
        for k2 in range(L):
            k = j * L + k2
            n = base + k
            t = tgtc[k2]
            c0 = pl.multiple_of(((t & 127) >> 4) << 4, 16)
            off = t & 15
            row = tile_v[k, n % 8, pl.ds(c0, L)]
            sel = jnp.where(lanes == off, row, zero)
            acc = acc + sel * weic[k2]
    pair_v[0, :] = acc
    pair_v[1, :] = cnt
    pltpu.sync_copy(pair_v, part_hbm.at[s])

    plsc.subcore_barrier()

    @pl.when(s == 0)
    def _finalize():
        pltpu.sync_copy(part_hbm, red_v)
        tot = jnp.zeros((L,), jnp.float32)
        ctot = jnp.zeros((L,), jnp.float32)
        for i in range(NS):
            tot = tot + red_v[i, 0]
            ctot = ctot + red_v[i, 1]

        buf_v[pl.ds(L, L)] = jnp.zeros((L,), jnp.float32)

        def lane_reduce(vec):
            cur = vec
            for off in (8, 4, 2, 1):
                buf_v[pl.ds(0, L)] = cur
                cur = cur + buf_v[pl.ds(off, L)]
            return cur

        vsum = lane_reduce(tot)
        csum = lane_reduce(ctot)
        out_v[...] = -(vsum / csum)
        pltpu.sync_copy(out_v, out_hbm)


def kernel(lprobs, target, seq_padding_mask, reward):
    B, S, V = lprobs.shape
    lp3 = lprobs.reshape(B * S // 8, 8, V)
    tgt = target.reshape(-1).astype(jnp.int32)
    maskf = seq_padding_mask.reshape(-1).astype(jnp.float32)
    rew = reward.reshape(-1)
    mesh = plsc.VectorSubcoreMesh(core_axis_name="c", subcore_axis_name="s",
                                  num_cores=1)
    run = functools.partial(
        pl.kernel,
        mesh=mesh,
        out_type=(
            jax.ShapeDtypeStruct((L,), jnp.float32),
            jax.ShapeDtypeStruct((NS, 2, L), jnp.float32),
        ),
        compiler_params=pltpu.CompilerParams(use_tc_tiling_on_sc=True),
        scratch_types=[
            pltpu.VMEM((TOK, 8, 128), jnp.float32),
            pltpu.VMEM((2, L), jnp.float32),
            pltpu.VMEM((NS, 2, L), jnp.float32),
            pltpu.VMEM((2 * L,), jnp.float32),
            pltpu.VMEM((L,), jnp.float32),
            pltpu.VMEM((TOK,), jnp.int32),
            pltpu.VMEM((TOK,), jnp.float32),
            pltpu.VMEM((TOK,), jnp.float32),
            [pltpu.SemaphoreType.DMA] * (NSEM + 1),
        ],
    )(functools.partial(_body, V))
    out, _ = run(lp3, tgt, maskf, rew)
    return out[0]
